# Initial kernel scaffold; baseline (speedup 1.0000x reference)
import jax
import jax.numpy as jnp
from jax.experimental import pallas as pl


def kernel(x, b0_init_w, b0_init_b, b0_t1_w, b0_t1_b, b0_t1_g, b0_t1_be, b0_t1_m, b0_t1_v, b0_t2_w, b0_t2_b, b0_t2_g, b0_t2_be, b0_t2_m, b0_t2_v, b1_init_w, b1_init_b, b1_t1_w, b1_t1_b, b1_t1_g, b1_t1_be, b1_t1_m, b1_t1_v, b1_t2_w, b1_t2_b, b1_t2_g, b1_t2_be, b1_t2_m, b1_t2_v, b2_init_w, b2_init_b, b2_t1_w, b2_t1_b, b2_t1_g, b2_t1_be, b2_t1_m, b2_t1_v, b2_t2_w, b2_t2_b, b2_t2_g, b2_t2_be, b2_t2_m, b2_t2_v, b3_init_w, b3_init_b, b3_t1_w, b3_t1_b, b3_t1_g, b3_t1_be, b3_t1_m, b3_t1_v, b3_t2_w, b3_t2_b, b3_t2_g, b3_t2_be, b3_t2_m, b3_t2_v, b4_init_w, b4_init_b, b4_t1_w, b4_t1_b, b4_t1_g, b4_t1_be, b4_t1_m, b4_t1_v, b4_t2_w, b4_t2_b, b4_t2_g, b4_t2_be, b4_t2_m, b4_t2_v, hm_w1, hm_b1, hm_w2, hm_b2, pf_w1, pf_b1, pf_w2, pf_b2):
    raise NotImplementedError("write your pallas kernel here")


# speedup vs baseline: 1.4758x; 1.4758x over previous
import functools

import jax
import jax.numpy as jnp
import numpy as np
from jax.experimental import pallas as pl
from jax.experimental.pallas import tpu as pltpu

_BN_EPS = 1e-5
_C = 128


def _ceil8(v):
    return (v + 7) // 8 * 8


def _layout(W, d):
    dW = d * W
    base = _ceil8(dW + d)
    return base, dW


def _build_paired(D, t, base, dW, d, HW):
    D[pl.ds(base - dW - d, dW + d), 0:128] = jnp.zeros((dW + d, _C), jnp.float32)
    D[pl.ds(base + HW, dW + d), 0:128] = jnp.zeros((dW + d, _C), jnp.float32)
    D[pl.ds(base - dW - d, _ceil8(d)), 128:256] = jnp.zeros(
        (_ceil8(d), _C), jnp.float32)
    D[pl.ds(base - dW + HW, 2 * dW + d), 128:256] = jnp.zeros(
        (2 * dW + d, _C), jnp.float32)
    D[pl.ds(base, HW), 0:128] = t
    D[pl.ds(base - dW, HW), 128:256] = t


def _conv3x3(D, base, mask_l, mask_r, wp_ref, ws_ref, b_ref, *, W, d, HW):
    dW = d * W
    acc = jnp.broadcast_to(b_ref[...], (HW, _C)).astype(jnp.float32)
    for kx in range(3):
        tap = D[pl.ds(base - dW + (kx - 1) * d, HW), :]
        if kx == 0:
            tap = jnp.where(mask_l, tap, 0.0)
        elif kx == 2:
            tap = jnp.where(mask_r, tap, 0.0)
        acc = acc + jnp.dot(tap, wp_ref[kx], preferred_element_type=jnp.float32)
    for kx in range(3):
        tap = D[pl.ds(base + dW + (kx - 1) * d, HW), 0:128]
        if kx == 0:
            tap = jnp.where(mask_l, tap, 0.0)
        elif kx == 2:
            tap = jnp.where(mask_r, tap, 0.0)
        acc = acc + jnp.dot(tap, ws_ref[kx], preferred_element_type=jnp.float32)
    return acc


def _stage_kernel(x_ref, col_ref, *refs, W, HW):
    (o_ref,) = refs[-3:-2]
    D1, D2 = refs[-2:]
    wrefs = refs[:-3]
    base1, dW1 = _layout(W, 1)
    base2, dW2 = _layout(W, 2)
    col = col_ref[...]
    m_l1, m_r1 = col >= 1, col < (W - 1)
    m_l2, m_r2 = col >= 2, col < (W - 2)

    x = x_ref[0]
    for b in range(5):
        w0, b0, wp1, ws1, b1, wp2, ws2, b2 = wrefs[8 * b:8 * b + 8]
        init = jnp.dot(x, w0[...], preferred_element_type=jnp.float32)
        init = jnp.maximum(init + b0[...], 0.0)
        _build_paired(D1, init, base1, dW1, 1, HW)
        t = _conv3x3(D1, base1, m_l1, m_r1, wp1, ws1, b1, W=W, d=1, HW=HW)
        t = jnp.maximum(t, 0.0)
        _build_paired(D2, t, base2, dW2, 2, HW)
        t = _conv3x3(D2, base2, m_l2, m_r2, wp2, ws2, b2, W=W, d=2, HW=HW)
        x = D1[pl.ds(base1, HW), 0:128] + jnp.maximum(t, 0.0)

    wh1, bh1, wh2, bh2 = wrefs[40:44]
    m = jnp.dot(x, wh1[...], preferred_element_type=jnp.float32)
    m = jnp.maximum(m + bh1[...], 0.0)
    o_ref[0] = jnp.dot(m, wh2[...], preferred_element_type=jnp.float32) + bh2[...]


def _fold_bn(w_oihw, b, g, be, mu, v):
    s = g / jnp.sqrt(v + _BN_EPS)
    return w_oihw * s[:, None, None, None], (b - mu) * s + be


def _io(w_oihw):
    return jnp.transpose(w_oihw[:, :, 0, 0], (1, 0))


def _tap_weights(w_oihw):
    w = jnp.transpose(w_oihw, (2, 3, 1, 0))
    wp = jnp.concatenate([w[0], w[1]], axis=1)
    return wp, w[2]


def kernel(x, b0_init_w, b0_init_b, b0_t1_w, b0_t1_b, b0_t1_g, b0_t1_be, b0_t1_m, b0_t1_v, b0_t2_w, b0_t2_b, b0_t2_g, b0_t2_be, b0_t2_m, b0_t2_v, b1_init_w, b1_init_b, b1_t1_w, b1_t1_b, b1_t1_g, b1_t1_be, b1_t1_m, b1_t1_v, b1_t2_w, b1_t2_b, b1_t2_g, b1_t2_be, b1_t2_m, b1_t2_v, b2_init_w, b2_init_b, b2_t1_w, b2_t1_b, b2_t1_g, b2_t1_be, b2_t1_m, b2_t1_v, b2_t2_w, b2_t2_b, b2_t2_g, b2_t2_be, b2_t2_m, b2_t2_v, b3_init_w, b3_init_b, b3_t1_w, b3_t1_b, b3_t1_g, b3_t1_be, b3_t1_m, b3_t1_v, b3_t2_w, b3_t2_b, b3_t2_g, b3_t2_be, b3_t2_m, b3_t2_v, b4_init_w, b4_init_b, b4_t1_w, b4_t1_b, b4_t1_g, b4_t1_be, b4_t1_m, b4_t1_v, b4_t2_w, b4_t2_b, b4_t2_g, b4_t2_be, b4_t2_m, b4_t2_v, hm_w1, hm_b1, hm_w2, hm_b2, pf_w1, pf_b1, pf_w2, pf_b2):
    N, Cin, H, W = x.shape
    HW = H * W
    cin_p = (Cin + 127) // 128 * 128
    n_hm, n_pf = hm_w2.shape[0], pf_w2.shape[0]

    blocks_raw = [
        (b0_init_w, b0_init_b, b0_t1_w, b0_t1_b, (b0_t1_g, b0_t1_be, b0_t1_m, b0_t1_v),
         b0_t2_w, b0_t2_b, (b0_t2_g, b0_t2_be, b0_t2_m, b0_t2_v)),
        (b1_init_w, b1_init_b, b1_t1_w, b1_t1_b, (b1_t1_g, b1_t1_be, b1_t1_m, b1_t1_v),
         b1_t2_w, b1_t2_b, (b1_t2_g, b1_t2_be, b1_t2_m, b1_t2_v)),
        (b2_init_w, b2_init_b, b2_t1_w, b2_t1_b, (b2_t1_g, b2_t1_be, b2_t1_m, b2_t1_v),
         b2_t2_w, b2_t2_b, (b2_t2_g, b2_t2_be, b2_t2_m, b2_t2_v)),
        (b3_init_w, b3_init_b, b3_t1_w, b3_t1_b, (b3_t1_g, b3_t1_be, b3_t1_m, b3_t1_v),
         b3_t2_w, b3_t2_b, (b3_t2_g, b3_t2_be, b3_t2_m, b3_t2_v)),
        (b4_init_w, b4_init_b, b4_t1_w, b4_t1_b, (b4_t1_g, b4_t1_be, b4_t1_m, b4_t1_v),
         b4_t2_w, b4_t2_b, (b4_t2_g, b4_t2_be, b4_t2_m, b4_t2_v)),
    ]

    wlist, wspecs = [], []

    def add_w(a):
        wlist.append(a)
        wspecs.append(
            pl.BlockSpec(a.shape, lambda b, nd=a.ndim: (0,) * nd))

    for i, (iw, ib, t1w, t1b, t1bn, t2w, t2b, t2bn) in enumerate(blocks_raw):
        w0 = _io(iw)
        if i == 0:
            w0 = jnp.pad(w0, ((0, cin_p - Cin), (0, 0)))
        t1w, t1b = _fold_bn(t1w, t1b, *t1bn)
        t2w, t2b = _fold_bn(t2w, t2b, *t2bn)
        wp1, ws1 = _tap_weights(t1w)
        wp2, ws2 = _tap_weights(t2w)
        add_w(w0)
        add_w(ib.reshape(1, -1))
        add_w(wp1)
        add_w(ws1)
        add_w(t1b.reshape(1, -1))
        add_w(wp2)
        add_w(ws2)
        add_w(t2b.reshape(1, -1))

    wh1 = jnp.concatenate([_io(hm_w1), _io(pf_w1)], axis=1)
    bh1 = jnp.concatenate([hm_b1, pf_b1]).reshape(1, -1)
    wh2 = jnp.zeros((2 * _C, 2 * _C), jnp.float32)
    wh2 = wh2.at[:_C, :n_hm].set(_io(hm_w2))
    wh2 = wh2.at[_C:, n_hm:n_hm + n_pf].set(_io(pf_w2))
    bh2 = jnp.zeros((1, 2 * _C), jnp.float32)
    bh2 = bh2.at[0, :n_hm].set(hm_b2)
    bh2 = bh2.at[0, n_hm:n_hm + n_pf].set(pf_b2)
    for a in (wh1, bh1, wh2, bh2):
        add_w(a)

    xp = jnp.transpose(x, (0, 2, 3, 1)).astype(jnp.float32).reshape(N, HW, Cin)
    xp = jnp.pad(xp, ((0, 0), (0, 0), (0, cin_p - Cin)))
    col = (jnp.arange(HW, dtype=jnp.int32) % W).reshape(HW, 1)

    base1, dW1 = _layout(W, 1)
    base2, dW2 = _layout(W, 2)
    L1 = _ceil8(base1 + dW1 + 1 + HW)
    L2 = _ceil8(base2 + dW2 + 2 + HW)

    out = pl.pallas_call(
        functools.partial(_stage_kernel, W=W, HW=HW),
        out_shape=jax.ShapeDtypeStruct((N, HW, 2 * _C), jnp.float32),
        grid=(N,),
        in_specs=[
            pl.BlockSpec((1, HW, cin_p), lambda b: (b, 0, 0)),
            pl.BlockSpec((HW, 1), lambda b: (0, 0)),
            *wspecs,
        ],
        out_specs=pl.BlockSpec((1, HW, 2 * _C), lambda b: (b, 0, 0)),
        scratch_shapes=[
            pltpu.VMEM((L1, 2 * _C), jnp.float32),
            pltpu.VMEM((L2, 2 * _C), jnp.float32),
        ],
        compiler_params=pltpu.CompilerParams(
            dimension_semantics=("parallel",)),
    )(xp, col, *wlist)

    hm = out[:, :, :n_hm].reshape(N, H, W, n_hm)
    pf = out[:, :, n_hm:n_hm + n_pf].reshape(N, H, W, n_pf)
    return [jnp.transpose(hm, (0, 3, 1, 2)), jnp.transpose(pf, (0, 3, 1, 2))]


# speedup vs baseline: 2.3761x; 1.6100x over previous
import functools

import jax
import jax.numpy as jnp
import numpy as np
from jax.experimental import pallas as pl
from jax.experimental.pallas import tpu as pltpu

_BN_EPS = 1e-5
_C = 128
_G = 4


def _ceil8(v):
    return (v + 7) // 8 * 8


def _geom(W):
    return _ceil8(W + _G + 2)


def _build_shifted(D, t, d, M, Wp, zb):
    dWp = d * Wp
    for kx in range(3):
        b = zb - (kx - 1) * d
        lo, hi = 128 * kx, 128 * (kx + 1)
        above = b - (zb - dWp)
        below = (zb + dWp + M) - (b + M)
        D[pl.ds(zb - dWp, above), lo:hi] = jnp.zeros((above, _C), jnp.float32)
        D[pl.ds(b + M, below), lo:hi] = jnp.zeros((below, _C), jnp.float32)
        D[pl.ds(b, M), lo:hi] = t


def _conv3x3(D, wp_ref, ws_ref, b_ref, *, Wp, d, M, zb):
    acc = jnp.broadcast_to(b_ref[...], (M, _C)).astype(jnp.float32)
    for ky in range(3):
        s = zb + (ky - 1) * d * Wp
        acc = acc + jnp.dot(D[pl.ds(s, M), 0:256], wp_ref[ky],
                            preferred_element_type=jnp.float32)
        acc = acc + jnp.dot(D[pl.ds(s, M), 256:384], ws_ref[ky],
                            preferred_element_type=jnp.float32)
    return acc


def _stage_kernel(x_ref, col_ref, *refs, Wp, W, M, zb):
    (o_ref,) = refs[-3:-2]
    D1, D2 = refs[-2:]
    wrefs = refs[:-3]
    col = col_ref[...]
    valid = (col >= _G) & (col < _G + W)

    x = x_ref[0]
    for b in range(5):
        w0, b0, wp1, ws1, b1, wp2, ws2, b2 = wrefs[8 * b:8 * b + 8]
        init = jnp.dot(x, w0[...], preferred_element_type=jnp.float32)
        init = jnp.where(valid, jnp.maximum(init + b0[...], 0.0), 0.0)
        _build_shifted(D1, init, 1, M, Wp, zb)
        t = _conv3x3(D1, wp1, ws1, b1, Wp=Wp, d=1, M=M, zb=zb)
        t = jnp.where(valid, jnp.maximum(t, 0.0), 0.0)
        _build_shifted(D2, t, 2, M, Wp, zb)
        t = _conv3x3(D2, wp2, ws2, b2, Wp=Wp, d=2, M=M, zb=zb)
        t = jnp.where(valid, jnp.maximum(t, 0.0), 0.0)
        x = D1[pl.ds(zb, M), 128:256] + t

    wh1, bh1, wh2, bh2 = wrefs[40:44]
    m = jnp.dot(x, wh1[...], preferred_element_type=jnp.float32)
    m = jnp.maximum(m + bh1[...], 0.0)
    o_ref[0] = jnp.dot(m, wh2[...], preferred_element_type=jnp.float32) + bh2[...]


def _fold_bn(w_oihw, b, g, be, mu, v):
    s = g / jnp.sqrt(v + _BN_EPS)
    return w_oihw * s[:, None, None, None], (b - mu) * s + be


def _io(w_oihw):
    return jnp.transpose(w_oihw[:, :, 0, 0], (1, 0))


def _tap_weights(w_oihw):
    w = jnp.transpose(w_oihw, (2, 3, 1, 0))
    wp = jnp.concatenate([w[:, 0], w[:, 1]], axis=1)
    return wp, w[:, 2]


def kernel(x, b0_init_w, b0_init_b, b0_t1_w, b0_t1_b, b0_t1_g, b0_t1_be, b0_t1_m, b0_t1_v, b0_t2_w, b0_t2_b, b0_t2_g, b0_t2_be, b0_t2_m, b0_t2_v, b1_init_w, b1_init_b, b1_t1_w, b1_t1_b, b1_t1_g, b1_t1_be, b1_t1_m, b1_t1_v, b1_t2_w, b1_t2_b, b1_t2_g, b1_t2_be, b1_t2_m, b1_t2_v, b2_init_w, b2_init_b, b2_t1_w, b2_t1_b, b2_t1_g, b2_t1_be, b2_t1_m, b2_t1_v, b2_t2_w, b2_t2_b, b2_t2_g, b2_t2_be, b2_t2_m, b2_t2_v, b3_init_w, b3_init_b, b3_t1_w, b3_t1_b, b3_t1_g, b3_t1_be, b3_t1_m, b3_t1_v, b3_t2_w, b3_t2_b, b3_t2_g, b3_t2_be, b3_t2_m, b3_t2_v, b4_init_w, b4_init_b, b4_t1_w, b4_t1_b, b4_t1_g, b4_t1_be, b4_t1_m, b4_t1_v, b4_t2_w, b4_t2_b, b4_t2_g, b4_t2_be, b4_t2_m, b4_t2_v, hm_w1, hm_b1, hm_w2, hm_b2, pf_w1, pf_b1, pf_w2, pf_b2):
    N, Cin, H, W = x.shape
    Wp = _geom(W)
    M = H * Wp
    cin_p = (Cin + 127) // 128 * 128
    n_hm, n_pf = hm_w2.shape[0], pf_w2.shape[0]

    blocks_raw = [
        (b0_init_w, b0_init_b, b0_t1_w, b0_t1_b, (b0_t1_g, b0_t1_be, b0_t1_m, b0_t1_v),
         b0_t2_w, b0_t2_b, (b0_t2_g, b0_t2_be, b0_t2_m, b0_t2_v)),
        (b1_init_w, b1_init_b, b1_t1_w, b1_t1_b, (b1_t1_g, b1_t1_be, b1_t1_m, b1_t1_v),
         b1_t2_w, b1_t2_b, (b1_t2_g, b1_t2_be, b1_t2_m, b1_t2_v)),
        (b2_init_w, b2_init_b, b2_t1_w, b2_t1_b, (b2_t1_g, b2_t1_be, b2_t1_m, b2_t1_v),
         b2_t2_w, b2_t2_b, (b2_t2_g, b2_t2_be, b2_t2_m, b2_t2_v)),
        (b3_init_w, b3_init_b, b3_t1_w, b3_t1_b, (b3_t1_g, b3_t1_be, b3_t1_m, b3_t1_v),
         b3_t2_w, b3_t2_b, (b3_t2_g, b3_t2_be, b3_t2_m, b3_t2_v)),
        (b4_init_w, b4_init_b, b4_t1_w, b4_t1_b, (b4_t1_g, b4_t1_be, b4_t1_m, b4_t1_v),
         b4_t2_w, b4_t2_b, (b4_t2_g, b4_t2_be, b4_t2_m, b4_t2_v)),
    ]

    wlist, wspecs = [], []

    def add_w(a):
        wlist.append(a)
        wspecs.append(
            pl.BlockSpec(a.shape, lambda b, nd=a.ndim: (0,) * nd))

    for i, (iw, ib, t1w, t1b, t1bn, t2w, t2b, t2bn) in enumerate(blocks_raw):
        w0 = _io(iw)
        if i == 0:
            w0 = jnp.pad(w0, ((0, cin_p - Cin), (0, 0)))
        t1w, t1b = _fold_bn(t1w, t1b, *t1bn)
        t2w, t2b = _fold_bn(t2w, t2b, *t2bn)
        wp1, ws1 = _tap_weights(t1w)
        wp2, ws2 = _tap_weights(t2w)
        add_w(w0)
        add_w(ib.reshape(1, -1))
        add_w(wp1)
        add_w(ws1)
        add_w(t1b.reshape(1, -1))
        add_w(wp2)
        add_w(ws2)
        add_w(t2b.reshape(1, -1))

    wh1 = jnp.concatenate([_io(hm_w1), _io(pf_w1)], axis=1)
    bh1 = jnp.concatenate([hm_b1, pf_b1]).reshape(1, -1)
    wh2 = jnp.zeros((2 * _C, 2 * _C), jnp.float32)
    wh2 = wh2.at[:_C, :n_hm].set(_io(hm_w2))
    wh2 = wh2.at[_C:, n_hm:n_hm + n_pf].set(_io(pf_w2))
    bh2 = jnp.zeros((1, 2 * _C), jnp.float32)
    bh2 = bh2.at[0, :n_hm].set(hm_b2)
    bh2 = bh2.at[0, n_hm:n_hm + n_pf].set(pf_b2)
    for a in (wh1, bh1, wh2, bh2):
        add_w(a)

    xp = jnp.transpose(x, (0, 2, 3, 1)).astype(jnp.float32)
    xp = jnp.pad(xp, ((0, 0), (0, 0), (_G, Wp - W - _G), (0, cin_p - Cin)))
    xp = xp.reshape(N, M, cin_p)
    col = (jnp.arange(M, dtype=jnp.int32) % Wp).reshape(M, 1)

    zb = _ceil8(2 * Wp + 2)
    LD = zb + 2 * Wp + M + 8

    out = pl.pallas_call(
        functools.partial(_stage_kernel, Wp=Wp, W=W, M=M, zb=zb),
        out_shape=jax.ShapeDtypeStruct((N, M, 2 * _C), jnp.float32),
        grid=(N,),
        in_specs=[
            pl.BlockSpec((1, M, cin_p), lambda b: (b, 0, 0)),
            pl.BlockSpec((M, 1), lambda b: (0, 0)),
            *wspecs,
        ],
        out_specs=pl.BlockSpec((1, M, 2 * _C), lambda b: (b, 0, 0)),
        scratch_shapes=[
            pltpu.VMEM((LD, 3 * _C), jnp.float32),
            pltpu.VMEM((LD, 3 * _C), jnp.float32),
        ],
        compiler_params=pltpu.CompilerParams(
            dimension_semantics=("parallel",)),
    )(xp, col, *wlist)

    outp = out.reshape(N, H, Wp, 2 * _C)[:, :, _G:_G + W, :]
    hm = outp[..., :n_hm]
    pf = outp[..., n_hm:n_hm + n_pf]
    return [jnp.transpose(hm, (0, 3, 1, 2)), jnp.transpose(pf, (0, 3, 1, 2))]
